# baseline (speedup 1.0000x reference)
import jax
import jax.numpy as jnp
from jax.experimental import pallas as pl
from jax.experimental.pallas import tpu as pltpu

K_DET = 64


def _lane_cumsum(x):
    n = x.shape[-1]
    sh = 1
    while sh < n:
        z = jnp.zeros(x.shape[:-1] + (sh,), x.dtype)
        x = x + jnp.concatenate([z, x[..., :-sh]], axis=-1)
        sh *= 2
    return x


def _search_max(pred, nbits):
    t = jnp.zeros((1, 1), jnp.int32)
    for b in range(nbits - 1, -1, -1):
        tp = jnp.bitwise_or(t, jnp.int32(1 << b))
        t = jnp.where(pred(tp), tp, t)
    return t


def _select64(vals, flat, nbits_flat):
    u = jax.lax.bitcast_convert_type(vals, jnp.int32)

    def pred_v(t):
        cnt = jnp.sum(jnp.where(u >= t, 1, 0), axis=(0, 1), keepdims=True)
        return cnt >= K_DET

    t1 = _search_max(pred_v, 30)
    gt = u > t1
    eq = u == t1
    g = jnp.sum(jnp.where(gt, 1, 0), axis=(0, 1), keepdims=True)
    need = K_DET - g

    def pred_f(fb):
        cnt = jnp.sum(jnp.where(eq & (flat < fb), 1, 0),
                      axis=(0, 1), keepdims=True)
        return cnt < need

    f1 = _search_max(pred_f, nbits_flat)
    return gt | (eq & (flat <= f1))


def _compact64(kept, tri_strict):
    ki = jnp.where(kept, 1, 0)
    incl = _lane_cumsum(ki)
    rowsum = incl[:, -1:].astype(jnp.float32)
    prefix = jax.lax.dot_general(tri_strict, rowsum, (((1,), (0,)), ((), ())),
                                 precision=jax.lax.Precision.HIGHEST)
    s = incl - ki + prefix.astype(jnp.int32)
    rl = kept.shape[0] * kept.shape[1]
    sm = jnp.where(kept, s, -1)
    sflat = sm.reshape(1, rl)
    qiota = jax.lax.broadcasted_iota(jnp.int32, (K_DET, 1), 0)
    sel = jnp.where(qiota == sflat, 1.0, 0.0)
    return sel


def _body(hm_ref, reg_ref, wh_ref, bb_ref, gv_ref):
    C, H, W = hm_ref.shape[1], hm_ref.shape[2], hm_ref.shape[3]
    hi = jax.lax.Precision.HIGHEST
    x = hm_ref[0]
    s = jax.nn.sigmoid(x)
    ninf = jnp.float32(-jnp.inf)
    padh = jnp.full((C, 1, W), ninf, jnp.float32)
    v = jnp.maximum(s, jnp.concatenate([s[:, 1:, :], padh], axis=1))
    v = jnp.maximum(v, jnp.concatenate([padh, s[:, :-1, :]], axis=1))
    padw = jnp.full((C, H, 1), ninf, jnp.float32)
    hmax = jnp.maximum(v, jnp.concatenate([v[:, :, 1:], padw], axis=2))
    hmax = jnp.maximum(hmax, jnp.concatenate([padw, v[:, :, :-1]], axis=2))
    nmsed = jnp.where(s == hmax, s, jnp.float32(0.0))
    l1 = jnp.max(nmsed, axis=2)

    R = C * H
    flat_ci = (jax.lax.broadcasted_iota(jnp.int32, (C, H), 0) * H
               + jax.lax.broadcasted_iota(jnp.int32, (C, H), 1))
    jiota = jax.lax.broadcasted_iota(jnp.int32, (1, W), 1)
    kiota = jax.lax.broadcasted_iota(jnp.int32, (1, K_DET), 1)
    riota64 = jax.lax.broadcasted_iota(jnp.int32, (K_DET, 1), 0)

    kept1 = _select64(l1, flat_ci, 14)
    triC = jnp.where(jax.lax.broadcasted_iota(jnp.int32, (C, C), 0)
                     > jax.lax.broadcasted_iota(jnp.int32, (C, C), 1),
                     1.0, 0.0)
    sel1 = _compact64(kept1, triC)
    nms2d = nmsed.reshape(R, W)
    cand = jax.lax.dot_general(sel1, nms2d, (((1,), (0,)), ((), ())),
                               precision=hi)
    rowidf = jax.lax.dot_general(flat_ci.astype(jnp.float32).reshape(1, R),
                                 sel1, (((1,), (1,)), ((), ())),
                                 precision=hi)
    rowid_col = jnp.round(rowidf).astype(jnp.int32).reshape(K_DET, 1)

    eflat = rowid_col * W + jiota
    kept2 = _select64(cand, eflat, 21)
    tri64 = jnp.where(
        riota64 > jax.lax.broadcasted_iota(jnp.int32, (K_DET, K_DET), 1),
        1.0, 0.0)
    sel2 = _compact64(kept2, tri64)
    EL = K_DET * W
    vals_row = jax.lax.dot_general(cand.reshape(1, EL), sel2,
                                   (((1,), (1,)), ((), ())), precision=hi)
    ids_row = jax.lax.dot_general(eflat.astype(jnp.float32).reshape(1, EL),
                                  sel2, (((1,), (1,)), ((), ())),
                                  precision=hi)

    vcol = vals_row.reshape(K_DET, 1)
    icol = ids_row.reshape(K_DET, 1)
    beats = ((vcol > vals_row)
             | ((vcol == vals_row) & (icol < ids_row)))
    rank_row = jnp.sum(jnp.where(beats, 1, 0), axis=0, keepdims=True)
    rank_col = rank_row.reshape(K_DET, 1)
    q = jnp.where(rank_col == kiota, 1.0, 0.0)
    vvec = jax.lax.dot_general(vals_row, q, (((1,), (0,)), ((), ())),
                               precision=hi)
    gvecf = jax.lax.dot_general(ids_row, q, (((1,), (0,)), ((), ())),
                                precision=hi)
    gvec = jnp.round(gvecf).astype(jnp.int32)
    gcol = gvec.reshape(K_DET, 1)

    z = jnp.zeros((1, K_DET), jnp.float32)
    r0a = r1a = w0a = w1a = z
    for k in range(K_DET):
        g = jnp.sum(jnp.where(kiota == k, gvec, 0))
        i = jax.lax.bitwise_and(jax.lax.shift_right_logical(g, 7),
                                jnp.int32(H - 1))
        j = jax.lax.bitwise_and(g, jnp.int32(W - 1))
        sh = jnp.int32(W) - j
        sel = kiota == k
        r0a = jnp.where(sel,
                        pltpu.roll(reg_ref[0, 0, pl.ds(i, 1), :], sh, 1)[:, 0:1],
                        r0a)
        r1a = jnp.where(sel,
                        pltpu.roll(reg_ref[0, 1, pl.ds(i, 1), :], sh, 1)[:, 0:1],
                        r1a)
        w0a = jnp.where(sel,
                        pltpu.roll(wh_ref[0, 0, pl.ds(i, 1), :], sh, 1)[:, 0:1],
                        w0a)
        w1a = jnp.where(sel,
                        pltpu.roll(wh_ref[0, 1, pl.ds(i, 1), :], sh, 1)[:, 0:1],
                        w1a)

    gv_ref[0] = gvec

    iv = jax.lax.bitwise_and(jax.lax.shift_right_logical(gvec, 7),
                             jnp.int32(H - 1))
    jv = jax.lax.bitwise_and(gvec, jnp.int32(W - 1))
    cf = jax.lax.shift_right_logical(gvec, 14).astype(jnp.float32)
    xc = jv.astype(jnp.float32) + r0a
    yc = iv.astype(jnp.float32) + r1a
    half = jnp.float32(0.5)
    bb_ref[0] = jnp.concatenate(
        [xc - w0a * half, yc - w1a * half, xc + w0a * half, yc + w1a * half,
         vvec, cf, jnp.zeros((2, K_DET), jnp.float32)],
        axis=0)


def _conv_body(gv_ref, conv_ref, cv_ref):
    CW, H, W = conv_ref.shape[1], conv_ref.shape[2], conv_ref.shape[3]
    pcol = jax.lax.bitwise_and(gv_ref[0], jnp.int32(H * W - 1))
    pcol = pcol.reshape(K_DET, 1)
    onehot = jnp.where(jax.lax.broadcasted_iota(jnp.int32, (K_DET, H * W), 1)
                       == pcol, 1.0, 0.0)
    convr = conv_ref[0].reshape(CW, H * W)
    cv_ref[0] = jax.lax.dot_general(onehot, convr, (((1,), (1,)), ((), ())),
                                    precision=jax.lax.Precision.HIGHEST)


def kernel(hm, reg, wh, seg_feat, conv_weight):
    B, C, H, W = hm.shape
    CW = conv_weight.shape[1]
    bb, gv = pl.pallas_call(
        _body,
        grid=(B,),
        in_specs=[
            pl.BlockSpec((1, C, H, W), lambda b: (b, 0, 0, 0)),
            pl.BlockSpec((1, 2, H, W), lambda b: (b, 0, 0, 0)),
            pl.BlockSpec((1, 2, H, W), lambda b: (b, 0, 0, 0)),
        ],
        out_specs=[
            pl.BlockSpec((1, 8, K_DET), lambda b: (b, 0, 0)),
            pl.BlockSpec((1, 1, K_DET), lambda b: (b, 0, 0)),
        ],
        out_shape=[
            jax.ShapeDtypeStruct((B, 8, K_DET), jnp.float32),
            jax.ShapeDtypeStruct((B, 1, K_DET), jnp.int32),
        ],
    )(hm, reg, wh)
    cv = pl.pallas_call(
        _conv_body,
        grid=(B,),
        in_specs=[
            pl.BlockSpec((1, 1, K_DET), lambda b: (b, 0, 0)),
            pl.BlockSpec((1, CW, H, W), lambda b: (b, 0, 0, 0)),
        ],
        out_specs=pl.BlockSpec((1, K_DET, CW), lambda b: (b, 0, 0)),
        out_shape=jax.ShapeDtypeStruct((B, K_DET, CW), jnp.float32),
    )(gv, conv_weight)
    bboxes = jnp.transpose(bb[:, 0:6, :], (0, 2, 1))
    return (bboxes, seg_feat, cv)
